# Initial kernel scaffold; baseline (speedup 1.0000x reference)
import functools

import jax
import jax.numpy as jnp
from jax import lax
from jax.experimental import pallas as pl
from jax.experimental.pallas import tpu as pltpu
from jax.experimental.pallas import tpu_sc as plsc

N = 100000
E = 300000
D = 128
NC = 2
NS = 16
HALF = N // NC

S = 18752
EPAD = NS * S
SV = S // 16

B = 128
CBUF = ((S + B - 1) // B) * B

CHUNK_OFF = (0, 12800, 25600, 38400)
CHUNK_SZ = (12800, 12800, 12800, 11600)
ACC_ROWS = 12808
DUMMY = 12800

BIAS_ROWS = 64
SENTINEL = jnp.int32(0x7FFFFFFF)


def _body(src_iu, dst_iu, tab_iu, src_ui, dst_ui, tab_ui, bias_hbm,
          out_user, out_item,
          src_stripe, dst_stripe, src_cbuf, dst_cbuf,
          src_blk, dst_blk, rows, bias_v, acc, sem):
  c = lax.axis_index("c")
  s = lax.axis_index("s")

  pltpu.sync_copy(bias_hbm, bias_v)

  def do_etype(table, src2d, dst2d, out):
    pltpu.sync_copy(src2d.at[s], src_stripe)
    pltpu.sync_copy(dst2d.at[s], dst_stripe)

    for off, ch in zip(CHUNK_OFF, CHUNK_SZ):
      lo = c * HALF + off
      hi = lo + ch
      rpt = ch // NS

      n_full, rem = divmod(rpt, BIAS_ROWS)
      for b in range(n_full):
        pltpu.sync_copy(bias_v, acc.at[pl.ds(s * rpt + b * BIAS_ROWS,
                                             BIAS_ROWS)])
      if rem:
        pltpu.sync_copy(bias_v.at[pl.ds(0, rem)],
                        acc.at[pl.ds(s * rpt + n_full * BIAS_ROWS, rem)])
      plsc.subcore_barrier()

      def scan_body(i, cnt):
        d = dst_stripe[pl.ds(i * 16, 16)]
        sv = src_stripe[pl.ds(i * 16, 16)]
        m = (d >= lo) & (d < hi)
        plsc.store_compressed(src_cbuf.at[pl.ds(cnt, 16)], sv, mask=m)
        plsc.store_compressed(dst_cbuf.at[pl.ds(cnt, 16)], d - lo, mask=m)
        return cnt + jnp.sum(m.astype(jnp.int32))

      cnt = lax.fori_loop(0, SV, scan_body, jnp.int32(0))

      def drain_body(j, _):
        base = j * B
        for i in range(B // 16):
          pos = base + i * 16 + lax.iota(jnp.int32, 16)
          mm = pos < cnt
          sv = jnp.where(mm, src_cbuf[pl.ds(base + i * 16, 16)], 0)
          dv = jnp.where(mm, dst_cbuf[pl.ds(base + i * 16, 16)],
                         jnp.int32(DUMMY))
          src_blk[pl.ds(i * 16, 16)] = sv
          dst_blk[pl.ds(i * 16, 16)] = dv
        pltpu.async_copy(table.at[src_blk], rows, sem).wait()
        pltpu.sync_copy(rows, acc.at[dst_blk], add=True)
        return 0

      nblk = (cnt + B - 1) // B
      lax.fori_loop(0, nblk, drain_body, 0)
      plsc.subcore_barrier()

      pltpu.sync_copy(acc.at[pl.ds(s * rpt, rpt)],
                      out.at[pl.ds(c * HALF + off + s * rpt, rpt)])

  do_etype(tab_iu, src_iu, dst_iu, out_user)
  do_etype(tab_ui, src_ui, dst_ui, out_item)


@jax.jit
def _run(src_iu, dst_iu, tab_iu, src_ui, dst_ui, tab_ui, bias_blk):
  mesh = plsc.VectorSubcoreMesh(core_axis_name="c", subcore_axis_name="s")
  f = pl.kernel(
      _body,
      out_type=(
          jax.ShapeDtypeStruct((N, D), jnp.float32),
          jax.ShapeDtypeStruct((N, D), jnp.float32),
      ),
      mesh=mesh,
      scratch_types=[
          pltpu.VMEM((S,), jnp.int32),
          pltpu.VMEM((S,), jnp.int32),
          pltpu.VMEM((CBUF,), jnp.int32),
          pltpu.VMEM((CBUF,), jnp.int32),
          pltpu.VMEM((B,), jnp.int32),
          pltpu.VMEM((B,), jnp.int32),
          pltpu.VMEM((B, D), jnp.float32),
          pltpu.VMEM((BIAS_ROWS, D), jnp.float32),
          pltpu.VMEM_SHARED((ACC_ROWS, D), jnp.float32),
          pltpu.SemaphoreType.DMA,
      ],
  )
  return f(src_iu, dst_iu, tab_iu, src_ui, dst_ui, tab_ui, bias_blk)


def _prep_edges(edge):
  src = jnp.pad(edge[0], (0, EPAD - E)).reshape(NS, S)
  dst = jnp.pad(edge[1], (0, EPAD - E),
                constant_values=SENTINEL).reshape(NS, S)
  return src, dst


def kernel(embed_u_u2i, embed_i_i2u, h_bias, edge_u2i, edge_i2u):
  src_iu, dst_iu = _prep_edges(edge_i2u)
  src_ui, dst_ui = _prep_edges(edge_u2i)
  bias_blk = jnp.broadcast_to(h_bias, (BIAS_ROWS, D))
  h_user, h_item = _run(src_iu, dst_iu, embed_i_i2u,
                        src_ui, dst_ui, embed_u_u2i, bias_blk)
  return (h_user, h_item)


# speedup vs baseline: 2.5182x; 2.5182x over previous
import jax
import jax.numpy as jnp
from jax import lax
from jax.experimental import pallas as pl
from jax.experimental.pallas import tpu as pltpu
from jax.experimental.pallas import tpu_sc as plsc

N = 100000
E = 300000
D = 128
NC = 2
NS = 16
HALF = N // NC

S = 18944
EPAD = NS * S
PIECE = 2368
NP = S // PIECE
PV = PIECE // 16

B = 16

CHUNK_OFF = (0, 12800, 25600, 38400)
CHUNK_SZ = (12800, 12800, 12800, 11600)
ACC_ROWS = 12808
DUMMY = 12800

BIAS_ROWS = 32
SENTINEL = 0x7FFFFFFF


def _body(src_iu, dst_iu, tab_iu, src_ui, dst_ui, tab_ui, bias_hbm,
          out_user, out_item,
          src_pc, dst_pc, src_cb, dst_cb, src_blk, dst_blk, rows, bias_v,
          src_blk2, dst_blk2, rows2, acc, sem):
  c = lax.axis_index("c")
  s = lax.axis_index("s")

  pltpu.sync_copy(bias_hbm, bias_v)

  def drain_block(table, sb, db, rw):
    for i in range(B // 16):
      sb[pl.ds(i * 16, 16)] = src_cb[pl.ds(i * 16, 16)]
      db[pl.ds(i * 16, 16)] = dst_cb[pl.ds(i * 16, 16)]
    pltpu.async_copy(table.at[sb], rw, sem).wait()
    pltpu.sync_copy(rw, acc.at[db], add=True)

  def do_etype(table, src2d, dst2d, out):
    def init_rows(start, size):
      n_full, rem = divmod(size, BIAS_ROWS)
      for b in range(n_full):
        pltpu.sync_copy(bias_v, acc.at[pl.ds(start + b * BIAS_ROWS,
                                             BIAS_ROWS)])
      if rem:
        pltpu.sync_copy(bias_v.at[pl.ds(0, rem)],
                        acc.at[pl.ds(start + n_full * BIAS_ROWS, rem)])

    def flush_rows(gbase, start, size):
      pltpu.sync_copy(acc.at[pl.ds(start, size)],
                      out.at[pl.ds(gbase + start, size)])

    for off, ch in zip(CHUNK_OFF, CHUNK_SZ):
      lo = c * HALF + off
      hi = lo + ch
      rpt_a = -(-(ch // NS) // 8) * 8
      rpt_b = ch - (NS - 1) * rpt_a
      ragged = rpt_b != rpt_a

      if not ragged:
        init_rows(s * rpt_a, rpt_a)
      else:
        pl.when(s < NS - 1)(lambda: init_rows(s * rpt_a, rpt_a))
        pl.when(s == NS - 1)(lambda: init_rows((NS - 1) * rpt_a, rpt_b))
      plsc.subcore_barrier()

      def scan_body(i, carry):
        fcnt, par = carry
        d = dst_pc[pl.ds(i * 16, 16)]
        sv = src_pc[pl.ds(i * 16, 16)]
        m = (d >= lo) & (d < hi)
        plsc.store_compressed(src_cb.at[pl.ds(fcnt, 16)], sv, mask=m)
        plsc.store_compressed(dst_cb.at[pl.ds(fcnt, 16)], d - lo, mask=m)
        fcnt = fcnt + jnp.sum(m.astype(jnp.int32))

        def drain_and_shift(sb, db, rw):
          def go():
            drain_block(table, sb, db, rw)
            src_cb[pl.ds(0, 16)] = src_cb[pl.ds(B, 16)]
            dst_cb[pl.ds(0, 16)] = dst_cb[pl.ds(B, 16)]
          return go

        full = fcnt >= B
        pl.when(full & (par == 0))(drain_and_shift(src_blk, dst_blk, rows))
        pl.when(full & (par == 1))(drain_and_shift(src_blk2, dst_blk2, rows2))
        par = jnp.where(full, 1 - par, par)
        return fcnt - jnp.where(full, B, 0), par

      def piece_body(p, carry):
        base = s * S + p * PIECE
        pltpu.sync_copy(src2d.at[pl.ds(base, PIECE)], src_pc)
        pltpu.sync_copy(dst2d.at[pl.ds(base, PIECE)], dst_pc)
        return lax.fori_loop(0, PV, scan_body, carry)

      fcnt, par = lax.fori_loop(0, NP, piece_body,
                                (jnp.int32(0), jnp.int32(0)))

      for i in range(B // 16):
        pos = i * 16 + lax.iota(jnp.int32, 16)
        mm = pos < fcnt
        src_cb[pl.ds(i * 16, 16)] = jnp.where(
            mm, src_cb[pl.ds(i * 16, 16)], pos & 63)
        dst_cb[pl.ds(i * 16, 16)] = jnp.where(
            mm, dst_cb[pl.ds(i * 16, 16)], DUMMY + (pos & 7))
      pl.when((fcnt > 0) & (par == 0))(
          lambda: drain_block(table, src_blk, dst_blk, rows))
      pl.when((fcnt > 0) & (par == 1))(
          lambda: drain_block(table, src_blk2, dst_blk2, rows2))
      plsc.subcore_barrier()

      gbase = c * HALF + off
      if not ragged:
        flush_rows(gbase, s * rpt_a, rpt_a)
      else:
        pl.when(s < NS - 1)(lambda: flush_rows(gbase, s * rpt_a, rpt_a))
        pl.when(s == NS - 1)(
            lambda: flush_rows(gbase, (NS - 1) * rpt_a, rpt_b))
      plsc.subcore_barrier()

  do_etype(tab_iu, src_iu, dst_iu, out_user)
  do_etype(tab_ui, src_ui, dst_ui, out_item)


@jax.jit
def _run(src_iu, dst_iu, tab_iu, src_ui, dst_ui, tab_ui, bias_blk):
  mesh = plsc.VectorSubcoreMesh(core_axis_name="c", subcore_axis_name="s")
  f = pl.kernel(
      _body,
      out_type=(
          jax.ShapeDtypeStruct((N, D), jnp.float32),
          jax.ShapeDtypeStruct((N, D), jnp.float32),
      ),
      mesh=mesh,
      compiler_params=pltpu.CompilerParams(needs_layout_passes=False),
      scratch_types=[
          pltpu.VMEM((PIECE,), jnp.int32),
          pltpu.VMEM((PIECE,), jnp.int32),
          pltpu.VMEM((B + 16,), jnp.int32),
          pltpu.VMEM((B + 16,), jnp.int32),
          pltpu.VMEM((B,), jnp.int32),
          pltpu.VMEM((B,), jnp.int32),
          pltpu.VMEM((B, D), jnp.float32),
          pltpu.VMEM((BIAS_ROWS, D), jnp.float32),
          pltpu.VMEM((B,), jnp.int32),
          pltpu.VMEM((B,), jnp.int32),
          pltpu.VMEM((B, D), jnp.float32),
          pltpu.VMEM_SHARED((ACC_ROWS, D), jnp.float32),
          pltpu.SemaphoreType.DMA,
      ],
  )
  return f(src_iu, dst_iu, tab_iu, src_ui, dst_ui, tab_ui, bias_blk)


def _prep_edges(edge):
  src = jnp.pad(edge[0], (0, EPAD - E))
  dst = jnp.pad(edge[1], (0, EPAD - E), constant_values=SENTINEL)
  return src, dst


def kernel(embed_u_u2i, embed_i_i2u, h_bias, edge_u2i, edge_i2u):
  src_iu, dst_iu = _prep_edges(edge_i2u)
  src_ui, dst_ui = _prep_edges(edge_u2i)
  bias_blk = jnp.broadcast_to(h_bias, (BIAS_ROWS, D))
  h_user, h_item = _run(src_iu, dst_iu, embed_i_i2u,
                        src_ui, dst_ui, embed_u_u2i, bias_blk)
  return (h_user, h_item)


# speedup vs baseline: 4.2063x; 1.6704x over previous
import jax
import jax.numpy as jnp
from jax import lax
from jax.experimental import pallas as pl
from jax.experimental.pallas import tpu as pltpu
from jax.experimental.pallas import tpu_sc as plsc

N = 100000
E = 300000
D = 128
NC = 2
NS = 16
HALF = N // NC

S = 18944
EPAD = NS * S
PIECE = 2368
NP = S // PIECE
PV = PIECE // 16

B = 128

CHUNK_OFF = (0, 12800, 25600, 38400)
CHUNK_SZ = (12800, 12800, 12800, 11600)
ACC_ROWS = 12808
DUMMY = 12800

BIAS_ROWS = 32
SENTINEL = 0x7FFFFFFF


def _body(src_iu, dst_iu, tab_iu, src_ui, dst_ui, tab_ui, bias_hbm,
          out_user, out_item,
          src_pc, dst_pc, src_cb, dst_cb, src_blk, dst_blk, rows, bias_v,
          acc, sem):
  c = lax.axis_index("c")
  s = lax.axis_index("s")

  pltpu.sync_copy(bias_hbm, bias_v)

  def drain_block(table, sb, db, rw):
    for i in range(B // 16):
      sb[pl.ds(i * 16, 16)] = src_cb[pl.ds(i * 16, 16)]
      db[pl.ds(i * 16, 16)] = dst_cb[pl.ds(i * 16, 16)]
    pltpu.async_copy(table.at[sb], rw, sem).wait()
    pltpu.sync_copy(rw, acc.at[db], add=True)

  def do_etype(table, src2d, dst2d, out):
    def init_rows(start, size):
      n_full, rem = divmod(size, BIAS_ROWS)
      for b in range(n_full):
        pltpu.sync_copy(bias_v, acc.at[pl.ds(start + b * BIAS_ROWS,
                                             BIAS_ROWS)])
      if rem:
        pltpu.sync_copy(bias_v.at[pl.ds(0, rem)],
                        acc.at[pl.ds(start + n_full * BIAS_ROWS, rem)])

    def flush_rows(gbase, start, size):
      pltpu.sync_copy(acc.at[pl.ds(start, size)],
                      out.at[pl.ds(gbase + start, size)])

    for off, ch in zip(CHUNK_OFF, CHUNK_SZ):
      lo = c * HALF + off
      hi = lo + ch
      rpt_a = -(-(ch // NS) // 8) * 8
      rpt_b = ch - (NS - 1) * rpt_a
      ragged = rpt_b != rpt_a

      if not ragged:
        init_rows(s * rpt_a, rpt_a)
      else:
        pl.when(s < NS - 1)(lambda: init_rows(s * rpt_a, rpt_a))
        pl.when(s == NS - 1)(lambda: init_rows((NS - 1) * rpt_a, rpt_b))
      plsc.subcore_barrier()

      def scan_body(i, fcnt):
        d = dst_pc[pl.ds(i * 16, 16)]
        sv = src_pc[pl.ds(i * 16, 16)]
        m = (d >= lo) & (d < hi)
        plsc.store_compressed(src_cb.at[pl.ds(fcnt, 16)], sv, mask=m)
        plsc.store_compressed(dst_cb.at[pl.ds(fcnt, 16)], d - lo, mask=m)
        fcnt = fcnt + jnp.sum(m.astype(jnp.int32))

        def drain_and_shift():
          drain_block(table, src_blk, dst_blk, rows)
          src_cb[pl.ds(0, 16)] = src_cb[pl.ds(B, 16)]
          dst_cb[pl.ds(0, 16)] = dst_cb[pl.ds(B, 16)]

        full = fcnt >= B
        pl.when(full)(drain_and_shift)
        return fcnt - jnp.where(full, B, 0)

      def piece_body(p, fcnt):
        base = s * S + p * PIECE
        pltpu.sync_copy(src2d.at[pl.ds(base, PIECE)], src_pc)
        pltpu.sync_copy(dst2d.at[pl.ds(base, PIECE)], dst_pc)
        return lax.fori_loop(0, PV, scan_body, fcnt)

      fcnt = lax.fori_loop(0, NP, piece_body, jnp.int32(0))

      for i in range(B // 16):
        pos = i * 16 + lax.iota(jnp.int32, 16)
        mm = pos < fcnt
        src_cb[pl.ds(i * 16, 16)] = jnp.where(
            mm, src_cb[pl.ds(i * 16, 16)], pos & 63)
        dst_cb[pl.ds(i * 16, 16)] = jnp.where(
            mm, dst_cb[pl.ds(i * 16, 16)], DUMMY + (pos & 7))
      pl.when(fcnt > 0)(
          lambda: drain_block(table, src_blk, dst_blk, rows))
      plsc.subcore_barrier()

      gbase = c * HALF + off
      if not ragged:
        flush_rows(gbase, s * rpt_a, rpt_a)
      else:
        pl.when(s < NS - 1)(lambda: flush_rows(gbase, s * rpt_a, rpt_a))
        pl.when(s == NS - 1)(
            lambda: flush_rows(gbase, (NS - 1) * rpt_a, rpt_b))
      plsc.subcore_barrier()

  do_etype(tab_iu, src_iu, dst_iu, out_user)
  do_etype(tab_ui, src_ui, dst_ui, out_item)


@jax.jit
def _run(src_iu, dst_iu, tab_iu, src_ui, dst_ui, tab_ui, bias_blk):
  mesh = plsc.VectorSubcoreMesh(core_axis_name="c", subcore_axis_name="s")
  f = pl.kernel(
      _body,
      out_type=(
          jax.ShapeDtypeStruct((N, D), jnp.float32),
          jax.ShapeDtypeStruct((N, D), jnp.float32),
      ),
      mesh=mesh,
      compiler_params=pltpu.CompilerParams(needs_layout_passes=False),
      scratch_types=[
          pltpu.VMEM((PIECE,), jnp.int32),
          pltpu.VMEM((PIECE,), jnp.int32),
          pltpu.VMEM((B + 16,), jnp.int32),
          pltpu.VMEM((B + 16,), jnp.int32),
          pltpu.VMEM((B,), jnp.int32),
          pltpu.VMEM((B,), jnp.int32),
          pltpu.VMEM((B, D), jnp.float32),
          pltpu.VMEM((BIAS_ROWS, D), jnp.float32),
          pltpu.VMEM_SHARED((ACC_ROWS, D), jnp.float32),
          pltpu.SemaphoreType.DMA,
      ],
  )
  return f(src_iu, dst_iu, tab_iu, src_ui, dst_ui, tab_ui, bias_blk)


def _prep_edges(edge):
  src = jnp.pad(edge[0], (0, EPAD - E))
  dst = jnp.pad(edge[1], (0, EPAD - E), constant_values=SENTINEL)
  return src, dst


def kernel(embed_u_u2i, embed_i_i2u, h_bias, edge_u2i, edge_i2u):
  src_iu, dst_iu = _prep_edges(edge_i2u)
  src_ui, dst_ui = _prep_edges(edge_u2i)
  bias_blk = jnp.broadcast_to(h_bias, (BIAS_ROWS, D))
  h_user, h_item = _run(src_iu, dst_iu, embed_i_i2u,
                        src_ui, dst_ui, embed_u_u2i, bias_blk)
  return (h_user, h_item)
